# baseline (speedup 1.0000x reference)
import functools
import math

import jax
import jax.numpy as jnp
from jax import lax
from jax.experimental import pallas as pl
from jax.experimental.pallas import tpu as pltpu
from jax.experimental.pallas import tpu_sc as plsc

_K = 2
_CF = 1.25
_MIN_CAP = 4
_E = 64
_T = 32768
_B = 2048
_NB = _T // _B
_CH = 128
_NCH = _B // _CH

_NW = 32
_BPW = _T // _NW
_L = 16


def _capacity(num_tokens, num_experts):
    cap = math.floor(_K * _CF * num_tokens / num_experts)
    cap += cap % 2
    return max(cap, _MIN_CAP)

_CAP = float(_capacity(_T, _E))


def _pass1_body(logits_ref, kept0_ref, v2_ref, r1_ref, i2_ref, c0_ref,
                aux_ref, carry, me_acc):
    i = pl.program_id(0)

    @pl.when(i == 0)
    def _init():
        carry[...] = jnp.zeros_like(carry)
        me_acc[...] = jnp.zeros_like(me_acc)

    r = jax.lax.broadcasted_iota(jnp.int32, (_CH, _CH), 0)
    c = jax.lax.broadcasted_iota(jnp.int32, (_CH, _CH), 1)
    triu = (r <= c).astype(jnp.bfloat16)
    iota_s = jax.lax.broadcasted_iota(jnp.int32, (_E, _CH), 0)

    running = carry[...]
    acc = me_acc[...]
    for j in range(_NCH):
        lt = logits_ref[pl.ds(j * _CH, _CH), :].T
        m = jnp.max(lt, axis=0, keepdims=True)
        ex = jnp.exp(lt - m)
        s = jnp.sum(ex, axis=0, keepdims=True)
        rs = 1.0 / s

        exi = jax.lax.bitcast_convert_type(ex, jnp.int32)
        key = (exi & jnp.int32(-64)) | (63 - iota_s)
        pm = jax.lax.bitcast_convert_type(key, jnp.float32)
        v1k = jnp.max(pm, axis=0, keepdims=True)
        oh0 = (pm == v1k)
        pm2 = jnp.where(oh0, 0.0, pm)
        v2k = jnp.max(pm2, axis=0, keepdims=True)
        oh1 = (pm2 == v2k)
        v1ki = jax.lax.bitcast_convert_type(v1k, jnp.int32)
        v2ki = jax.lax.bitcast_convert_type(v2k, jnp.int32)
        i2 = 63 - (v2ki & 63)
        val1 = jax.lax.bitcast_convert_type(
            v1ki & jnp.int32(-64), jnp.float32) * rs
        val2 = jax.lax.bitcast_convert_type(
            v2ki & jnp.int32(-64), jnp.float32) * rs

        a = jnp.concatenate([oh0.astype(jnp.bfloat16),
                             oh1.astype(jnp.bfloat16)], axis=0)
        cj = jnp.dot(a, triu, preferred_element_type=jnp.float32)
        cfull = cj + running
        prod = cfull * a.astype(jnp.float32)
        pos0 = jnp.sum(prod[:_E, :], axis=0, keepdims=True) - 1.0
        r1 = jnp.sum(prod[_E:, :], axis=0, keepdims=True) - 1.0
        running = running + cj[:, _CH - 1:_CH]

        keep0 = (pos0 < _CAP).astype(jnp.float32)
        sl = pl.ds(j * _CH, _CH)
        kept0_ref[:, :, sl] = (val1 * keep0).reshape(1, 1, _CH)
        v2_ref[:, :, sl] = val2.reshape(1, 1, _CH)
        r1_ref[:, :, sl] = r1.reshape(1, 1, _CH)
        i2_ref[:, :, sl] = i2.astype(jnp.float32).reshape(1, 1, _CH)

        acc = acc + ex * rs

    carry[...] = running
    me_acc[...] = acc

    @pl.when(i == _NB - 1)
    def _tail():
        new_c0 = running[:_E, :]
        c0_ref[...] = new_c0
        t = jnp.float32(_T)
        me_tot = jnp.sum(acc, axis=1, keepdims=True)
        aux_ref[...] = (jnp.float32(_E) * jnp.sum(
            (me_tot / t) * (new_c0 / t))).reshape(1, 1)


def _pass2_sc(kept0_hbm, v2_hbm, r1_hbm, i2_hbm, c0_hbm,
              out0_hbm, out1_hbm,
              kept0_v, v2_v, r1_v, i2_v, c0_v, out0_v, out1_v):
    wid = lax.axis_index("s") * 2 + lax.axis_index("c")
    base = wid * _BPW
    pltpu.sync_copy(kept0_hbm.at[pl.ds(base, _BPW)], kept0_v)
    pltpu.sync_copy(v2_hbm.at[pl.ds(base, _BPW)], v2_v)
    pltpu.sync_copy(r1_hbm.at[pl.ds(base, _BPW)], r1_v)
    pltpu.sync_copy(i2_hbm.at[pl.ds(base, _BPW)], i2_v)
    pltpu.sync_copy(c0_hbm, c0_v.at[pl.ds(0, _E)])
    c0q = [c0_v[pl.ds(16 * q, 16)] for q in range(4)]
    for g in range(_BPW // _L):
        sl = pl.ds(g * _L, _L)
        i2g = i2_v[sl].astype(jnp.int32)
        lo = i2g & 15
        hi = i2g >> 4
        gq = [c0q[q].at[lo].get(mode="promise_in_bounds")
              for q in range(4)]
        c0sel = jnp.where(hi == 0, gq[0],
                          jnp.where(hi == 1, gq[1],
                                    jnp.where(hi == 2, gq[2], gq[3])))
        r1g = r1_v[sl]
        kept0 = kept0_v[sl]
        keep1 = (c0sel + r1g) < _CAP
        kv1 = jnp.where(keep1, v2_v[sl], 0.0)
        denom = kept0 + kv1 + 1e-9
        out0_v[sl] = kept0 / denom
        out1_v[sl] = kv1 / denom
    pltpu.sync_copy(out0_v, out0_hbm.at[pl.ds(base, _BPW)])
    pltpu.sync_copy(out1_v, out1_hbm.at[pl.ds(base, _BPW)])


@jax.jit
def kernel(logits):
    tok_spec = pl.BlockSpec((1, 1, _B), lambda i: (i, 0, 0))
    tok_shape = jax.ShapeDtypeStruct((_NB, 1, _B), jnp.float32)
    kept0, v2, r1, i2, c0, aux = pl.pallas_call(
        _pass1_body,
        grid=(_NB,),
        in_specs=[pl.BlockSpec((_B, _E), lambda i: (i, 0))],
        out_specs=[tok_spec, tok_spec, tok_spec, tok_spec,
                   pl.BlockSpec((_E, 1), lambda i: (0, 0)),
                   pl.BlockSpec((1, 1), lambda i: (0, 0))],
        out_shape=[
            tok_shape, tok_shape, tok_shape, tok_shape,
            jax.ShapeDtypeStruct((_E, 1), jnp.float32),
            jax.ShapeDtypeStruct((1, 1), jnp.float32),
        ],
        scratch_shapes=[pltpu.VMEM((2 * _E, 1), jnp.float32),
                        pltpu.VMEM((_E, _CH), jnp.float32)],
    )(logits)

    mesh = plsc.VectorSubcoreMesh(core_axis_name="c", subcore_axis_name="s")
    pass2 = functools.partial(
        pl.kernel,
        mesh=mesh,
        out_type=[jax.ShapeDtypeStruct((_T,), jnp.float32)] * 2,
        scratch_types=[pltpu.VMEM((_BPW,), jnp.float32)] * 4
        + [pltpu.VMEM((128,), jnp.float32)]
        + [pltpu.VMEM((_BPW,), jnp.float32)] * 2,
    )(_pass2_sc)
    out0, out1 = pass2(kept0.reshape(-1), v2.reshape(-1), r1.reshape(-1),
                       i2.reshape(-1), c0.reshape(-1))

    combine = jnp.stack([out0, out1], axis=1)
    return combine, aux[0, 0]
